# baseline (speedup 1.0000x reference)
import functools
import jax
import jax.numpy as jnp
from jax import lax
from jax.experimental import pallas as pl
from jax.experimental.pallas import tpu as pltpu
from jax.experimental.pallas import tpu_sc as plsc

MAX_POS = 4096
HIDDEN = 1024
B = 4

info = plsc.get_sparse_core_info()
NC, NS = info.num_cores, info.num_subcores
NW = NC * NS
ROWS_PER_W = MAX_POS // NW
CHUNK = 64


def _sc_body(table_hbm, out_hbm, buf):
    wid = lax.axis_index("s") * NC + lax.axis_index("c")
    base = wid * ROWS_PER_W
    for c in range(ROWS_PER_W // CHUNK):
        r = base + c * CHUNK
        pltpu.sync_copy(table_hbm.at[pl.ds(r, CHUNK), :], buf)
        for b in range(B):
            pltpu.sync_copy(buf, out_hbm.at[b, pl.ds(r, CHUNK), :])


def kernel(position_ids, position_embeddings):
    Bd, S, H = position_ids.shape
    mesh = plsc.VectorSubcoreMesh(core_axis_name="c", subcore_axis_name="s")
    k = functools.partial(
        pl.kernel,
        mesh=mesh,
        out_type=jax.ShapeDtypeStruct((Bd, S, H), jnp.float32),
        scratch_types=[pltpu.VMEM((CHUNK, H), jnp.float32)],
    )(_sc_body)
    return k(position_embeddings[:S])
